# initial kernel scaffold (unmeasured)
import functools

import jax
import jax.numpy as jnp
from jax import lax
from jax.experimental import pallas as pl
from jax.experimental.pallas import tpu as pltpu

N_DEV = 4
SQ = 1024
SKV = 1024
HQ = 8
DH = 128
D = HQ * DH
BLK = 64
SCALE = 0.08838834764831843


def _body(x_ref, wq_ref, kv_ref, wo_ref, out_ref,
          kvfull, comm, qbuf, ctxbuf, send_sems, recv_sems):
    my = lax.axis_index("i")
    left = (my - 1) % N_DEV
    right = (my + 1) % N_DEV

    barrier_sem = pltpu.get_barrier_semaphore()
    for nbr in [left, right]:
        pl.semaphore_signal(
            barrier_sem, inc=1,
            device_id=(nbr,), device_id_type=pl.DeviceIdType.MESH,
        )
    pl.semaphore_wait(barrier_sem, 2)

    kvfull[pl.ds(my * SKV, SKV), :] = kv_ref[:, :]
    comm[0, :, :] = kv_ref[:, :]

    qbuf[:, :] = lax.dot_general(
        x_ref[:, :], wq_ref[:, :],
        (((1,), (0,)), ((), ())),
        preferred_element_type=jnp.float32,
    ).astype(jnp.bfloat16)

    for h in range(N_DEV - 1):
        send_slot = h % 2
        recv_slot = (h + 1) % 2
        rdma = pltpu.make_async_remote_copy(
            src_ref=comm.at[send_slot],
            dst_ref=comm.at[recv_slot],
            send_sem=send_sems.at[send_slot],
            recv_sem=recv_sems.at[recv_slot],
            device_id=(right,),
            device_id_type=pl.DeviceIdType.MESH,
        )
        rdma.start()
        rdma.wait()
        origin = (my - h - 1) % N_DEV
        kvfull[pl.ds(origin * SKV, SKV), :] = comm[recv_slot, :, :]

    kb = lax.broadcasted_iota(jnp.int32, (SQ, N_DEV * SKV), 1) // BLK
    qb = my * (SQ // BLK) + lax.broadcasted_iota(
        jnp.int32, (SQ, N_DEV * SKV), 0) // BLK
    mask = kb <= qb
    for h in range(HQ):
        qh = qbuf[:, h * DH:(h + 1) * DH]
        s_parts = []
        for j in range(N_DEV):
            kj = kvfull[j * SKV:(j + 1) * SKV, h * DH:(h + 1) * DH]
            s_parts.append(lax.dot_general(
                qh, kj, (((1,), (1,)), ((), ())),
                preferred_element_type=jnp.float32,
            ))
        s = jnp.concatenate(s_parts, axis=1) * SCALE
        s = jnp.where(mask, s, -1e9)
        m = jnp.max(s, axis=1, keepdims=True)
        w = jnp.exp(s - m)
        w = w / jnp.sum(w, axis=1, keepdims=True)
        wb = w.astype(jnp.bfloat16)
        ctx_h = lax.dot_general(
            wb[:, 0:SKV], kvfull[0:SKV, D + h * DH:D + (h + 1) * DH],
            (((1,), (0,)), ((), ())), preferred_element_type=jnp.float32)
        for j in range(1, N_DEV):
            ctx_h = ctx_h + lax.dot_general(
                wb[:, j * SKV:(j + 1) * SKV],
                kvfull[j * SKV:(j + 1) * SKV, D + h * DH:D + (h + 1) * DH],
                (((1,), (0,)), ((), ())), preferred_element_type=jnp.float32)
        ctxbuf[:, h * DH:(h + 1) * DH] = ctx_h.astype(jnp.bfloat16)

    out_ref[:, :] = lax.dot_general(
        ctxbuf[:, :], wo_ref[:, :],
        (((1,), (0,)), ((), ())),
        preferred_element_type=jnp.float32,
    )


def kernel(x, Wq, K_ext, V_ext, Wo):
    x2 = x[0].astype(jnp.bfloat16)
    wq = Wq.astype(jnp.bfloat16)
    wo = Wo.astype(jnp.bfloat16)
    kv = jnp.concatenate(
        [K_ext[0].reshape(SKV, D), V_ext[0].reshape(SKV, D)], axis=1
    ).astype(jnp.bfloat16)

    out = pl.pallas_call(
        _body,
        out_shape=jax.ShapeDtypeStruct((SQ, D), jnp.float32),
        in_specs=[
            pl.BlockSpec(memory_space=pltpu.VMEM),
            pl.BlockSpec(memory_space=pltpu.VMEM),
            pl.BlockSpec(memory_space=pltpu.VMEM),
            pl.BlockSpec(memory_space=pltpu.VMEM),
        ],
        out_specs=pl.BlockSpec(memory_space=pltpu.VMEM),
        scratch_shapes=[
            pltpu.VMEM((N_DEV * SKV, 2 * D), jnp.bfloat16),
            pltpu.VMEM((2, SKV, 2 * D), jnp.bfloat16),
            pltpu.VMEM((SQ, D), jnp.bfloat16),
            pltpu.VMEM((SQ, D), jnp.bfloat16),
            pltpu.SemaphoreType.DMA((2,)),
            pltpu.SemaphoreType.DMA((2,)),
        ],
        compiler_params=pltpu.CompilerParams(collective_id=0),
    )(x2, wq, kv, wo)
    return out[None]


# baseline (device time: 272555 ns/iter reference)
import functools

import jax
import jax.numpy as jnp
from jax import lax
from jax.experimental import pallas as pl
from jax.experimental.pallas import tpu as pltpu

N_DEV = 4
SQ = 1024
SKV = 1024
HQ = 8
DH = 128
D = HQ * DH
BLK = 64
SCALE = 0.08838834764831843


def _body(x_ref, wq_ref, kv_ref, wo_ref, out_ref,
          kvfull, qbuf, ctxbuf, send_sems, recv_sems):
    my = lax.axis_index("i")
    left = (my - 1) % N_DEV
    right = (my + 1) % N_DEV

    barrier_sem = pltpu.get_barrier_semaphore()
    for nbr in [left, right]:
        pl.semaphore_signal(
            barrier_sem, inc=1,
            device_id=(nbr,), device_id_type=pl.DeviceIdType.MESH,
        )
    pl.semaphore_wait(barrier_sem, 2)

    kvfull[pl.ds(my * SKV, SKV), :] = kv_ref[:, :]

    qbuf[:, :] = lax.dot_general(
        x_ref[:, :], wq_ref[:, :],
        (((1,), (0,)), ((), ())),
        preferred_element_type=jnp.float32,
    ).astype(jnp.bfloat16)

    for h in range(N_DEV - 1):
        origin_s = (my - h) % N_DEV
        origin_r = (my - h - 1) % N_DEV
        rdma = pltpu.make_async_remote_copy(
            src_ref=kvfull.at[pl.ds(origin_s * SKV, SKV), :],
            dst_ref=kvfull.at[pl.ds(origin_s * SKV, SKV), :],
            send_sem=send_sems.at[h % 2],
            recv_sem=recv_sems.at[h % 2],
            device_id=(right,),
            device_id_type=pl.DeviceIdType.MESH,
        )
        rdma.start()
        rdma.wait()

    QT = 128
    kb = lax.broadcasted_iota(jnp.int32, (QT, N_DEV * SKV), 1) // BLK
    for h in range(HQ):
        for t in range(SQ // QT):
            qh = qbuf[t * QT:(t + 1) * QT, h * DH:(h + 1) * DH]
            s_parts = []
            for j in range(N_DEV):
                kj = kvfull[j * SKV:(j + 1) * SKV, h * DH:(h + 1) * DH]
                s_parts.append(lax.dot_general(
                    qh, kj, (((1,), (1,)), ((), ())),
                    preferred_element_type=jnp.float32,
                ))
            s = jnp.concatenate(s_parts, axis=1) * SCALE
            qb = (my * SQ + t * QT + lax.broadcasted_iota(
                jnp.int32, (QT, N_DEV * SKV), 0)) // BLK
            s = jnp.where(kb <= qb, s, -1e9)
            m = jnp.max(s, axis=1, keepdims=True)
            w = jnp.exp(s - m)
            w = w / jnp.sum(w, axis=1, keepdims=True)
            wb = w.astype(jnp.bfloat16)
            ctx_h = lax.dot_general(
                wb[:, 0:SKV], kvfull[0:SKV, D + h * DH:D + (h + 1) * DH],
                (((1,), (0,)), ((), ())), preferred_element_type=jnp.float32)
            for j in range(1, N_DEV):
                ctx_h = ctx_h + lax.dot_general(
                    wb[:, j * SKV:(j + 1) * SKV],
                    kvfull[j * SKV:(j + 1) * SKV, D + h * DH:D + (h + 1) * DH],
                    (((1,), (0,)), ((), ())), preferred_element_type=jnp.float32)
            ctxbuf[t * QT:(t + 1) * QT, h * DH:(h + 1) * DH] = (
                ctx_h.astype(jnp.bfloat16))

    out_ref[:, :] = lax.dot_general(
        ctxbuf[:, :], wo_ref[:, :],
        (((1,), (0,)), ((), ())),
        preferred_element_type=jnp.float32,
    )


def kernel(x, Wq, K_ext, V_ext, Wo):
    x2 = x[0].astype(jnp.bfloat16)
    wq = Wq.astype(jnp.bfloat16)
    wo = Wo.astype(jnp.bfloat16)
    kv = jnp.concatenate(
        [K_ext[0].reshape(SKV, D), V_ext[0].reshape(SKV, D)], axis=1
    ).astype(jnp.bfloat16)

    out = pl.pallas_call(
        _body,
        out_shape=jax.ShapeDtypeStruct((SQ, D), jnp.float32),
        in_specs=[
            pl.BlockSpec(memory_space=pltpu.VMEM),
            pl.BlockSpec(memory_space=pltpu.VMEM),
            pl.BlockSpec(memory_space=pltpu.VMEM),
            pl.BlockSpec(memory_space=pltpu.VMEM),
        ],
        out_specs=pl.BlockSpec(memory_space=pltpu.VMEM),
        scratch_shapes=[
            pltpu.VMEM((N_DEV * SKV, 2 * D), jnp.bfloat16),
            pltpu.VMEM((SQ, D), jnp.bfloat16),
            pltpu.VMEM((SQ, D), jnp.bfloat16),
            pltpu.SemaphoreType.DMA((2,)),
            pltpu.SemaphoreType.DMA((2,)),
        ],
        compiler_params=pltpu.CompilerParams(
            collective_id=0, vmem_limit_bytes=60 * 1024 * 1024),
    )(x2, wq, kv, wo)
    return out[None]


# device time: 156870 ns/iter; 1.7375x vs baseline; 1.7375x over previous
import jax
import jax.numpy as jnp
from jax import lax
from jax.experimental import pallas as pl
from jax.experimental.pallas import tpu as pltpu

N_DEV = 4
SQ = 1024
SKV = 1024
HQ = 8
DH = 128
D = HQ * DH
BLK = 64
SCALE = 0.08838834764831843
NEG = -1e9


def _body(x_ref, wq_ref, kv_ref, wo_ref, out_ref, kvfull, qbuf,
          acc_ref, m_ref, l_ref, send_r, recv_r, send_l, recv_l):
    my = lax.axis_index("i")
    left = (my - 1) % N_DEV
    right = (my + 1) % N_DEV

    barrier_sem = pltpu.get_barrier_semaphore()
    for nbr in [left, right]:
        pl.semaphore_signal(
            barrier_sem, inc=1,
            device_id=(nbr,), device_id_type=pl.DeviceIdType.MESH,
        )
    pl.semaphore_wait(barrier_sem, 2)

    kvfull[pl.ds(my * SKV, SKV), :] = kv_ref[:, :]

    def chunk_region(origin):
        return kvfull.at[pl.ds(origin * SKV, SKV), :]

    rdma_r0 = pltpu.make_async_remote_copy(
        src_ref=chunk_region(my), dst_ref=chunk_region(my),
        send_sem=send_r.at[0], recv_sem=recv_r.at[0],
        device_id=(right,), device_id_type=pl.DeviceIdType.MESH,
    )
    rdma_r0.start()
    rdma_l0 = pltpu.make_async_remote_copy(
        src_ref=chunk_region(my), dst_ref=chunk_region(my),
        send_sem=send_l.at[0], recv_sem=recv_l.at[0],
        device_id=(left,), device_id_type=pl.DeviceIdType.MESH,
    )
    rdma_l0.start()

    qbuf[:, :] = lax.dot_general(
        x_ref[:, :], wq_ref[:, :],
        (((1,), (0,)), ((), ())),
        preferred_element_type=jnp.float32,
    ).astype(jnp.bfloat16)

    QT = 512
    kb_loc = lax.broadcasted_iota(jnp.int32, (1, SKV), 1) // BLK
    qb_loc = lax.broadcasted_iota(jnp.int32, (QT, 1), 0) // BLK

    def accum_chunk(origin, first=False):
        kb = origin * (SKV // BLK) + kb_loc
        for t in range(SQ // QT):
            rows = slice(t * QT, (t + 1) * QT)
            qb = my * (SQ // BLK) + t * (QT // BLK) + qb_loc
            mask = kb <= qb
            for h in range(HQ):
                qh = qbuf[rows, h * DH:(h + 1) * DH]
                kh = kvfull[pl.ds(origin * SKV, SKV), h * DH:(h + 1) * DH]
                vh = kvfull[pl.ds(origin * SKV, SKV),
                            D + h * DH:D + (h + 1) * DH]
                s = lax.dot_general(
                    qh, kh, (((1,), (1,)), ((), ())),
                    preferred_element_type=jnp.float32,
                ) * SCALE
                s = jnp.where(mask, s, NEG)
                if first:
                    m_new = jnp.max(s, axis=1, keepdims=True)
                    p = jnp.exp(s - m_new)
                    l_new = jnp.sum(p, axis=1, keepdims=True)
                    acc_new = lax.dot_general(
                        p.astype(jnp.bfloat16), vh, (((1,), (0,)), ((), ())),
                        preferred_element_type=jnp.float32,
                    )
                else:
                    m_old = m_ref[rows, h:h + 1]
                    m_new = jnp.maximum(
                        m_old, jnp.max(s, axis=1, keepdims=True))
                    p = jnp.exp(s - m_new)
                    alpha = jnp.exp(m_old - m_new)
                    l_new = alpha * l_ref[rows, h:h + 1] + jnp.sum(
                        p, axis=1, keepdims=True)
                    acc_new = alpha * acc_ref[rows, h * DH:(h + 1) * DH] + (
                        lax.dot_general(
                            p.astype(jnp.bfloat16), vh,
                            (((1,), (0,)), ((), ())),
                            preferred_element_type=jnp.float32,
                        ))
                m_ref[rows, h:h + 1] = m_new
                l_ref[rows, h:h + 1] = l_new
                acc_ref[rows, h * DH:(h + 1) * DH] = acc_new

    accum_chunk(my, first=True)

    rdma_r0.wait_recv()
    rdma_r1 = pltpu.make_async_remote_copy(
        src_ref=chunk_region(left), dst_ref=chunk_region(left),
        send_sem=send_r.at[1], recv_sem=recv_r.at[1],
        device_id=(right,), device_id_type=pl.DeviceIdType.MESH,
    )
    rdma_r1.start()
    accum_chunk(left)

    rdma_l0.wait_recv()
    accum_chunk(right)

    rdma_r1.wait_recv()
    accum_chunk((my - 2) % N_DEV)

    ctx = jnp.concatenate(
        [(acc_ref[:, h * DH:(h + 1) * DH] / l_ref[:, h:h + 1]
          ).astype(jnp.bfloat16) for h in range(HQ)], axis=1)
    out_ref[:, :] = lax.dot_general(
        ctx, wo_ref[:, :],
        (((1,), (0,)), ((), ())),
        preferred_element_type=jnp.float32,
    )

    rdma_r0.wait_send()
    rdma_l0.wait_send()
    rdma_r1.wait_send()


def kernel(x, Wq, K_ext, V_ext, Wo):
    x2 = x[0].astype(jnp.bfloat16)
    wq = Wq.astype(jnp.bfloat16)
    wo = Wo.astype(jnp.bfloat16)
    kv = jnp.concatenate(
        [K_ext[0].reshape(SKV, D), V_ext[0].reshape(SKV, D)], axis=1
    ).astype(jnp.bfloat16)

    out = pl.pallas_call(
        _body,
        out_shape=jax.ShapeDtypeStruct((SQ, D), jnp.float32),
        in_specs=[
            pl.BlockSpec(memory_space=pltpu.VMEM),
            pl.BlockSpec(memory_space=pltpu.VMEM),
            pl.BlockSpec(memory_space=pltpu.VMEM),
            pl.BlockSpec(memory_space=pltpu.VMEM),
        ],
        out_specs=pl.BlockSpec(memory_space=pltpu.VMEM),
        scratch_shapes=[
            pltpu.VMEM((N_DEV * SKV, 2 * D), jnp.bfloat16),
            pltpu.VMEM((SQ, D), jnp.bfloat16),
            pltpu.VMEM((SQ, D), jnp.float32),
            pltpu.VMEM((SQ, HQ), jnp.float32),
            pltpu.VMEM((SQ, HQ), jnp.float32),
            pltpu.SemaphoreType.DMA((2,)),
            pltpu.SemaphoreType.DMA((2,)),
            pltpu.SemaphoreType.DMA((1,)),
            pltpu.SemaphoreType.DMA((1,)),
        ],
        compiler_params=pltpu.CompilerParams(
            collective_id=0, vmem_limit_bytes=60 * 1024 * 1024),
    )(x2, wq, kv, wo)
    return out[None]
